# baseline (speedup 1.0000x reference)
import functools

import jax
import jax.numpy as jnp
from jax.experimental import pallas as pl
from jax.experimental.pallas import tpu as pltpu

D_MODEL = 1024
HIDDEN = 1024
NUM_EXPERTS = 64
TOKENS = 128
HBLK = 256
NHB = HIDDEN // HBLK


def _moe_body(x_ref, ws1_ref, ws2_ref, scp_ref, w1_ref, w2_ref, cp_ref,
              wr_ref, br_ref, out_ref,
              m1_ref, m2_ref, i1_ref, i2_ref, den_ref):
    hb = pl.program_id(0)
    e = pl.program_id(1)
    xv = x_ref[...]

    @pl.when((hb == 0) & (e == 0))
    def _router_and_init():
        logits = jnp.dot(xv, wr_ref[...], preferred_element_type=jnp.float32)
        logits = logits + br_ref[...]
        probs = jax.nn.softmax(logits, axis=-1)
        iota = jax.lax.broadcasted_iota(jnp.int32, probs.shape, 1)
        m1 = jnp.max(probs, axis=-1, keepdims=True)
        i1 = jnp.min(jnp.where(probs == m1, iota, NUM_EXPERTS),
                     axis=-1, keepdims=True)
        p2 = jnp.where(iota == i1, -jnp.inf, probs)
        m2 = jnp.max(p2, axis=-1, keepdims=True)
        i2 = jnp.min(jnp.where(p2 == m2, iota, NUM_EXPERTS),
                     axis=-1, keepdims=True)
        m1_ref[...] = m1
        m2_ref[...] = m2
        i1_ref[...] = i1
        i2_ref[...] = i2
        den_ref[...] = m1 + m2
        out_ref[...] = jnp.zeros_like(out_ref)

    g = (jnp.where(i1_ref[...] == e, m1_ref[...], 0.0)
         + jnp.where(i2_ref[...] == e, m2_ref[...], 0.0)) / den_ref[...]

    h = jax.nn.silu(jnp.dot(xv, w1_ref[0], preferred_element_type=jnp.float32))
    h = h * jnp.dot(xv, w2_ref[0], preferred_element_type=jnp.float32)
    out_ref[...] += jnp.dot(h * g, cp_ref[0], preferred_element_type=jnp.float32)

    @pl.when(e == 0)
    def _shared():
        sh = jax.nn.silu(jnp.dot(xv, ws1_ref[...],
                                 preferred_element_type=jnp.float32))
        sh = sh * jnp.dot(xv, ws2_ref[...], preferred_element_type=jnp.float32)
        out_ref[...] += jnp.dot(sh, scp_ref[...],
                                preferred_element_type=jnp.float32)


@jax.jit
def kernel(x, Ws1, Ws2, Scp, W1, W2, Cp, Wr, br):
    br2 = br.reshape(1, NUM_EXPERTS)
    grid = (NHB, NUM_EXPERTS)
    out = pl.pallas_call(
        _moe_body,
        grid=grid,
        in_specs=[
            pl.BlockSpec((TOKENS, D_MODEL), lambda hb, e: (0, 0)),
            pl.BlockSpec((D_MODEL, HBLK), lambda hb, e: (0, hb)),
            pl.BlockSpec((D_MODEL, HBLK), lambda hb, e: (0, hb)),
            pl.BlockSpec((HBLK, D_MODEL), lambda hb, e: (hb, 0)),
            pl.BlockSpec((1, D_MODEL, HBLK), lambda hb, e: (e, 0, hb)),
            pl.BlockSpec((1, D_MODEL, HBLK), lambda hb, e: (e, 0, hb)),
            pl.BlockSpec((1, HBLK, D_MODEL), lambda hb, e: (e, hb, 0)),
            pl.BlockSpec((D_MODEL, NUM_EXPERTS), lambda hb, e: (0, 0)),
            pl.BlockSpec((1, NUM_EXPERTS), lambda hb, e: (0, 0)),
        ],
        out_specs=pl.BlockSpec((TOKENS, D_MODEL), lambda hb, e: (0, 0)),
        out_shape=jax.ShapeDtypeStruct((TOKENS, D_MODEL), jnp.float32),
        scratch_shapes=[
            pltpu.VMEM((TOKENS, 1), jnp.float32),
            pltpu.VMEM((TOKENS, 1), jnp.float32),
            pltpu.VMEM((TOKENS, 1), jnp.int32),
            pltpu.VMEM((TOKENS, 1), jnp.int32),
            pltpu.VMEM((TOKENS, 1), jnp.float32),
        ],
        compiler_params=pltpu.CompilerParams(
            dimension_semantics=("arbitrary", "arbitrary"),
        ),
    )(x, Ws1, Ws2, Scp, W1, W2, Cp, Wr, br2)
    return out


# speedup vs baseline: 1.4333x; 1.4333x over previous
import functools

import jax
import jax.numpy as jnp
from jax.experimental import pallas as pl
from jax.experimental.pallas import tpu as pltpu

D_MODEL = 1024
HIDDEN = 1024
NUM_EXPERTS = 64
TOKENS = 128
HBLK = 1024
NHB = HIDDEN // HBLK


def _moe_body(x_ref, ws1_ref, ws2_ref, scp_ref, w1_ref, w2_ref, cp_ref,
              wr_ref, br_ref, out_ref,
              m1_ref, m2_ref, i1_ref, i2_ref, den_ref):
    hb = pl.program_id(0)
    e = pl.program_id(1)
    xv = x_ref[...]

    @pl.when((hb == 0) & (e == 0))
    def _router_and_init():
        logits = jnp.dot(xv, wr_ref[...], preferred_element_type=jnp.float32)
        logits = logits + br_ref[...]
        probs = jax.nn.softmax(logits, axis=-1)
        iota = jax.lax.broadcasted_iota(jnp.int32, probs.shape, 1)
        m1 = jnp.max(probs, axis=-1, keepdims=True)
        i1 = jnp.min(jnp.where(probs == m1, iota, NUM_EXPERTS),
                     axis=-1, keepdims=True)
        p2 = jnp.where(iota == i1, -jnp.inf, probs)
        m2 = jnp.max(p2, axis=-1, keepdims=True)
        i2 = jnp.min(jnp.where(p2 == m2, iota, NUM_EXPERTS),
                     axis=-1, keepdims=True)
        m1_ref[...] = m1
        m2_ref[...] = m2
        i1_ref[...] = i1
        i2_ref[...] = i2
        den_ref[...] = m1 + m2
        out_ref[...] = jnp.zeros_like(out_ref)

    g = (jnp.where(i1_ref[...] == e, m1_ref[...], 0.0)
         + jnp.where(i2_ref[...] == e, m2_ref[...], 0.0)) / den_ref[...]

    h = jax.nn.silu(jnp.dot(xv, w1_ref[0], preferred_element_type=jnp.float32))
    h = h * jnp.dot(xv, w2_ref[0], preferred_element_type=jnp.float32)
    out_ref[...] += jnp.dot(h * g, cp_ref[0], preferred_element_type=jnp.float32)

    @pl.when(e == 0)
    def _shared():
        sh = jax.nn.silu(jnp.dot(xv, ws1_ref[...],
                                 preferred_element_type=jnp.float32))
        sh = sh * jnp.dot(xv, ws2_ref[...], preferred_element_type=jnp.float32)
        out_ref[...] += jnp.dot(sh, scp_ref[...],
                                preferred_element_type=jnp.float32)


@jax.jit
def kernel(x, Ws1, Ws2, Scp, W1, W2, Cp, Wr, br):
    br2 = br.reshape(1, NUM_EXPERTS)
    grid = (NHB, NUM_EXPERTS)
    out = pl.pallas_call(
        _moe_body,
        grid=grid,
        in_specs=[
            pl.BlockSpec((TOKENS, D_MODEL), lambda hb, e: (0, 0)),
            pl.BlockSpec((D_MODEL, HBLK), lambda hb, e: (0, hb)),
            pl.BlockSpec((D_MODEL, HBLK), lambda hb, e: (0, hb)),
            pl.BlockSpec((HBLK, D_MODEL), lambda hb, e: (hb, 0)),
            pl.BlockSpec((1, D_MODEL, HBLK), lambda hb, e: (e, 0, hb)),
            pl.BlockSpec((1, D_MODEL, HBLK), lambda hb, e: (e, 0, hb)),
            pl.BlockSpec((1, HBLK, D_MODEL), lambda hb, e: (e, hb, 0)),
            pl.BlockSpec((D_MODEL, NUM_EXPERTS), lambda hb, e: (0, 0)),
            pl.BlockSpec((1, NUM_EXPERTS), lambda hb, e: (0, 0)),
        ],
        out_specs=pl.BlockSpec((TOKENS, D_MODEL), lambda hb, e: (0, 0)),
        out_shape=jax.ShapeDtypeStruct((TOKENS, D_MODEL), jnp.float32),
        scratch_shapes=[
            pltpu.VMEM((TOKENS, 1), jnp.float32),
            pltpu.VMEM((TOKENS, 1), jnp.float32),
            pltpu.VMEM((TOKENS, 1), jnp.int32),
            pltpu.VMEM((TOKENS, 1), jnp.int32),
            pltpu.VMEM((TOKENS, 1), jnp.float32),
        ],
        compiler_params=pltpu.CompilerParams(
            dimension_semantics=("arbitrary", "arbitrary"),
        ),
    )(x, Ws1, Ws2, Scp, W1, W2, Cp, Wr, br2)
    return out
